# Initial kernel scaffold; baseline (speedup 1.0000x reference)
import functools

import jax
import jax.numpy as jnp
from jax import lax
from jax.experimental import pallas as pl
from jax.experimental.pallas import tpu as pltpu
from jax.experimental.pallas import tpu_sc as plsc

D = 32
G = 128
NBUF = 4


@functools.cache
def _make(total_rows: int):
    info = plsc.get_sparse_core_info()
    nc, ns = info.num_cores, info.num_subcores
    nw = nc * ns
    assert total_rows % (nw * G) == 0
    n_groups = total_rows // (nw * G)
    assert n_groups % NBUF == 0

    mesh = plsc.VectorSubcoreMesh(core_axis_name="c", subcore_axis_name="s")

    @functools.partial(
        pl.kernel,
        mesh=mesh,
        out_type=jax.ShapeDtypeStruct((total_rows, D), jnp.float32),
        scratch_types=[
            pltpu.VMEM((n_groups, G), jnp.int32),
            *[pltpu.VMEM((G, D), jnp.float32) for _ in range(NBUF)],
            *[pltpu.SemaphoreType.DMA for _ in range(NBUF)],
        ],
    )
    def gather_kernel(table, idx, out, idx_v, *rest):
        rows = rest[:NBUF]
        gsem = rest[NBUF:]
        wid = lax.axis_index("s") * nc + lax.axis_index("c")
        base_g = wid * n_groups
        base_r = base_g * G

        pltpu.sync_copy(idx.at[pl.ds(base_g, n_groups)], idx_v)

        for b in range(NBUF):
            pltpu.async_copy(table.at[idx_v.at[b]], rows[b], gsem[b])

        def step(j0, carry):
            for b in range(NBUF):
                j = j0 * NBUF + b
                pltpu.make_async_copy(
                    table.at[idx_v.at[j]], rows[b], gsem[b]).wait()
                pltpu.sync_copy(rows[b], out.at[pl.ds(base_r + j * G, G)])
                nj = j + NBUF

                @pl.when(nj < n_groups)
                def _():
                    pltpu.async_copy(
                        table.at[idx_v.at[nj]], rows[b], gsem[b])
            return carry

        lax.fori_loop(0, n_groups // NBUF, step, 0)

    return gather_kernel


def kernel(token_ids, time_embeddings_param):
    b, t = token_ids.shape
    total = b * t
    idx2d = token_ids.astype(jnp.int32).reshape(total // G, G)
    out = _make(total)(time_embeddings_param, idx2d)
    return out.reshape(b, t, D)


# speedup vs baseline: 1.4950x; 1.4950x over previous
import functools

import jax
import jax.numpy as jnp
from jax import lax
from jax.experimental import pallas as pl
from jax.experimental.pallas import tpu as pltpu
from jax.experimental.pallas import tpu_sc as plsc

D = 32
G = 128
NBUF = 4


@functools.cache
def _make(total_rows: int):
    info = plsc.get_sparse_core_info()
    nc, ns = info.num_cores, info.num_subcores
    nw = nc * ns
    assert total_rows % (nw * G) == 0
    n_groups = total_rows // (nw * G)
    assert n_groups % NBUF == 0

    mesh = plsc.VectorSubcoreMesh(core_axis_name="c", subcore_axis_name="s")

    @functools.partial(
        pl.kernel,
        mesh=mesh,
        out_type=jax.ShapeDtypeStruct((total_rows, D), jnp.float32),
        compiler_params=pltpu.CompilerParams(use_tc_tiling_on_sc=False),
        scratch_types=[
            pltpu.VMEM((n_groups, G), jnp.int32),
            *[pltpu.VMEM((G, D), jnp.float32) for _ in range(NBUF)],
            *[pltpu.SemaphoreType.DMA for _ in range(NBUF)],
        ],
    )
    def gather_kernel(table, idx, out, idx_v, *rest):
        rows = rest[:NBUF]
        gsem = rest[NBUF:]
        wid = lax.axis_index("s") * nc + lax.axis_index("c")
        base_g = wid * n_groups
        base_r = base_g * G

        pltpu.sync_copy(idx.at[pl.ds(base_g, n_groups)], idx_v)

        for b in range(NBUF):
            pltpu.async_copy(table.at[idx_v.at[b]], rows[b], gsem[b])

        def step(j0, carry):
            for b in range(NBUF):
                j = j0 * NBUF + b
                pltpu.make_async_copy(
                    table.at[idx_v.at[j]], rows[b], gsem[b]).wait()
                pltpu.sync_copy(rows[b], out.at[pl.ds(base_r + j * G, G)])
                nj = j + NBUF

                @pl.when(nj < n_groups)
                def _():
                    pltpu.async_copy(
                        table.at[idx_v.at[nj]], rows[b], gsem[b])
            return carry

        lax.fori_loop(0, n_groups // NBUF, step, 0)

    return gather_kernel


def kernel(token_ids, time_embeddings_param):
    b, t = token_ids.shape
    total = b * t
    idx2d = token_ids.astype(jnp.int32).reshape(total // G, G)
    out = _make(total)(time_embeddings_param, idx2d)
    return out.reshape(b, t, D)
